# baseline (speedup 1.0000x reference)
import functools

import jax
import jax.numpy as jnp
from jax import lax
from jax.experimental import pallas as pl
from jax.experimental.pallas import tpu as pltpu
from jax.experimental.pallas import tpu_sc as plsc

EXPERTS = 8
D = 768
OUT = 768
N_TOK = 4096
NPAIR = 2 * N_TOK
MB = 256
NB = (NPAIR + EXPERTS * (MB - 1) + MB - 1) // MB
RPAD = NB * MB
CHUNK = 128
N_WORKERS = 32


def _route_kernel(xg_ref, gw_ref, gb_ref, w1_ref, w2_ref, pos_ref, be_ref,
                  ep_ref, cs_ref):
    xg = xg_ref[...]
    gates = lax.dot_general(
        xg, gw_ref[...], (((1,), (1,)), ((), ())),
        preferred_element_type=jnp.float32) + gb_ref[...][None, :]

    idx = lax.broadcasted_iota(jnp.int32, gates.shape, 1)
    v1 = jnp.max(gates, axis=-1, keepdims=True)
    i1 = jnp.min(jnp.where(gates == v1, idx, EXPERTS), axis=-1, keepdims=True)
    masked = jnp.where(idx == i1, -jnp.inf, gates)
    v2 = jnp.max(masked, axis=-1, keepdims=True)
    i2 = jnp.min(jnp.where(masked == v2, idx, EXPERTS), axis=-1, keepdims=True)
    t = jnp.exp(v2 - v1)
    w1_ref[...] = 1.0 / (1.0 + t)
    w2_ref[...] = t / (1.0 + t)

    ep_ref[pl.ds(0, N_TOK), :] = i1
    ep_ref[pl.ds(N_TOK, N_TOK), :] = i2

    csz = 512
    nch = NPAIR // csz
    iota_e = lax.broadcasted_iota(jnp.int32, (csz, EXPERTS), 1)

    ltri = (lax.broadcasted_iota(jnp.int32, (csz, csz), 0) >=
            lax.broadcasted_iota(jnp.int32, (csz, csz), 1)
            ).astype(jnp.float32)
    carry = jnp.zeros((1, EXPERTS), jnp.float32)
    for i in range(nch):
        ch = ep_ref[pl.ds(i * csz, csz), :]
        oh = (ch == iota_e).astype(jnp.float32)
        cs = lax.dot_general(ltri, oh, (((1,), (0,)), ((), ())),
                             preferred_element_type=jnp.float32) + carry
        cs_ref[pl.ds(i * csz, csz), :] = cs
        carry = cs[csz - 1:csz, :]

    counts = carry
    cap = ((counts.astype(jnp.int32) + MB - 1) // MB) * MB
    capf = cap.astype(jnp.float32)
    sl = (lax.broadcasted_iota(jnp.int32, (EXPERTS, EXPERTS), 0) <
          lax.broadcasted_iota(jnp.int32, (EXPERTS, EXPERTS), 1))
    pstartf = lax.dot_general(capf, sl.astype(jnp.float32),
                              (((1,), (0,)), ((), ())),
                              preferred_element_type=jnp.float32)
    pstart = pstartf.astype(jnp.int32)

    for i in range(nch):
        ch = ep_ref[pl.ds(i * csz, csz), :]
        oh = (ch == iota_e).astype(jnp.float32)
        cs = cs_ref[pl.ds(i * csz, csz), :]
        posf = jnp.sum(oh * (pstartf + cs), axis=1, keepdims=True) - 1.0
        pos_ref[pl.ds(i * csz, csz), :] = posf.astype(jnp.int32)

    brow = lax.broadcasted_iota(jnp.int32, (CHUNK, EXPERTS), 0) * MB
    iota_be = lax.broadcasted_iota(jnp.int32, (CHUNK, EXPERTS), 1)
    inseg = (brow >= pstart) & (brow < pstart + cap)
    be = jnp.sum(jnp.where(inseg, iota_be, 0), axis=1, keepdims=True)
    total_used = pstart[:, EXPERTS - 1:] + cap[:, EXPERTS - 1:]
    last_e = jnp.max(jnp.where(cap > 0, iota_be[:1], 0), axis=1,
                     keepdims=True)
    valid = brow[:, :1] < total_used
    be_ref[...] = jnp.where(valid, be, EXPERTS + last_e)


@functools.partial(jax.jit, static_argnames=("interpret",))
def _route(xg, gate_W, gate_b, interpret=False):
    return pl.pallas_call(
        _route_kernel,
        out_shape=[
            jax.ShapeDtypeStruct((N_TOK, 1), jnp.float32),
            jax.ShapeDtypeStruct((N_TOK, 1), jnp.float32),
            jax.ShapeDtypeStruct((NPAIR, 1), jnp.int32),
            jax.ShapeDtypeStruct((CHUNK, 1), jnp.int32),
        ],
        scratch_shapes=[pltpu.VMEM((NPAIR, 1), jnp.int32),
                        pltpu.VMEM((NPAIR, EXPERTS), jnp.float32)],
        interpret=interpret,
    )(xg, gate_W, gate_b)


def _ffn_kernel(be_ref, xs_ref, w0_ref, b0_ref, w1_ref, b1_ref, w2_ref,
                b2_ref, ys_ref):
    @pl.when(be_ref[pl.program_id(0)] < EXPERTS)
    def _():
        x = xs_ref[...]
        h = lax.dot_general(x, w0_ref[0], (((1,), (1,)), ((), ())),
                            preferred_element_type=jnp.float32)
        h = jnp.maximum(h + b0_ref[0], 0.0)
        h = lax.dot_general(h, w1_ref[0], (((1,), (1,)), ((), ())),
                            preferred_element_type=jnp.float32)
        h = jnp.maximum(h + b1_ref[0], 0.0)
        o = lax.dot_general(h, w2_ref[0], (((1,), (1,)), ((), ())),
                            preferred_element_type=jnp.float32)
        ys_ref[...] = o + b2_ref[0]


@functools.partial(jax.jit, static_argnames=("interpret",))
def _ffn(be, xs, W0, b0, W1, b1, W2, b2, interpret=False):
    grid_spec = pltpu.PrefetchScalarGridSpec(
        num_scalar_prefetch=1,
        grid=(NB,),
        in_specs=[
            pl.BlockSpec((MB, D), lambda b, be: (b, 0)),
            pl.BlockSpec((1, D, D), lambda b, be: (be[b] % EXPERTS, 0, 0)),
            pl.BlockSpec((1, 1, D), lambda b, be: (be[b] % EXPERTS, 0, 0)),
            pl.BlockSpec((1, D, D), lambda b, be: (be[b] % EXPERTS, 0, 0)),
            pl.BlockSpec((1, 1, D), lambda b, be: (be[b] % EXPERTS, 0, 0)),
            pl.BlockSpec((1, OUT, D), lambda b, be: (be[b] % EXPERTS, 0, 0)),
            pl.BlockSpec((1, 1, OUT), lambda b, be: (be[b] % EXPERTS, 0, 0)),
        ],
        out_specs=pl.BlockSpec((MB, OUT), lambda b, be: (b, 0)),
    )
    return pl.pallas_call(
        _ffn_kernel,
        grid_spec=grid_spec,
        out_shape=jax.ShapeDtypeStruct((RPAD, OUT), jnp.float32),
        compiler_params=pltpu.CompilerParams(
            dimension_semantics=("arbitrary",)),
        interpret=interpret,
    )(be, xs, W0, b0.reshape(EXPERTS, 1, D), W1, b1.reshape(EXPERTS, 1, D),
      W2, b2.reshape(EXPERTS, 1, OUT))


def _combine_kernel(y1_ref, y2_ref, w1_ref, w2_ref, out_ref):
    out_ref[...] = (y1_ref[...] * w1_ref[...] + y2_ref[...] * w2_ref[...])


@functools.partial(jax.jit, static_argnames=("interpret",))
def _combine(yw, w1, w2, interpret=False):
    mc = 1024
    nbc = N_TOK // mc
    return pl.pallas_call(
        _combine_kernel,
        grid=(nbc,),
        in_specs=[
            pl.BlockSpec((mc, OUT), lambda b: (b, 0)),
            pl.BlockSpec((mc, OUT), lambda b, nbc=nbc: (b + nbc, 0)),
            pl.BlockSpec((mc, 1), lambda b: (b, 0)),
            pl.BlockSpec((mc, 1), lambda b: (b, 0)),
        ],
        out_specs=pl.BlockSpec((mc, OUT), lambda b: (b, 0)),
        out_shape=jax.ShapeDtypeStruct((N_TOK, OUT), jnp.float32),
        interpret=interpret,
    )(yw, yw, w1, w2)


def _sc_mesh():
    return plsc.VectorSubcoreMesh(core_axis_name="c", subcore_axis_name="s")


@jax.jit
def _dispatch(x_flat, pos):
    @functools.partial(
        pl.kernel,
        out_type=jax.ShapeDtypeStruct((RPAD, D), jnp.float32),
        mesh=_sc_mesh(),
        scratch_types=[
            pltpu.VMEM((CHUNK,), jnp.int32),
            pltpu.VMEM((CHUNK, D), jnp.float32),
            pltpu.SemaphoreType.DMA,
        ],
    )
    def disp(x_hbm, pos_hbm, xs_hbm, idx_v, buf_v, sem):
        wid = lax.axis_index("s") * 2 + lax.axis_index("c")
        per_w = NPAIR // N_WORKERS

        @pl.loop(0, per_w // CHUNK)
        def _(c):
            p0 = wid * per_w + c * CHUNK
            t0 = lax.rem(p0, N_TOK)
            pltpu.sync_copy(pos_hbm.at[pl.ds(p0, CHUNK)], idx_v)
            pltpu.sync_copy(x_hbm.at[pl.ds(t0, CHUNK)], buf_v)
            pltpu.async_copy(buf_v, xs_hbm.at[idx_v], sem).wait()

    return disp(x_flat, pos)


@jax.jit
def _gather_back(ys, pos):
    @functools.partial(
        pl.kernel,
        out_type=jax.ShapeDtypeStruct((NPAIR, OUT), jnp.float32),
        mesh=_sc_mesh(),
        scratch_types=[
            pltpu.VMEM((CHUNK,), jnp.int32),
            pltpu.VMEM((CHUNK, OUT), jnp.float32),
            pltpu.SemaphoreType.DMA,
        ],
    )
    def gat(ys_hbm, pos_hbm, yw_hbm, idx_v, buf_v, sem):
        wid = lax.axis_index("s") * 2 + lax.axis_index("c")
        per_w = NPAIR // N_WORKERS

        @pl.loop(0, per_w // CHUNK)
        def _(c):
            p0 = wid * per_w + c * CHUNK
            pltpu.sync_copy(pos_hbm.at[pl.ds(p0, CHUNK)], idx_v)
            pltpu.async_copy(ys_hbm.at[idx_v], buf_v, sem).wait()
            pltpu.sync_copy(buf_v, yw_hbm.at[pl.ds(p0, CHUNK)])

    return gat(ys, pos)


def kernel(x, gate_W, gate_b, W0, b0, W1, b1, W2, b2):
    bsz, num_pairs, feat = x.shape
    x_flat = x.reshape(-1, feat)
    xg = x_flat[:, feat - 3:]
    w1, w2, pos, be = _route(xg, gate_W, gate_b)
    pos1 = pos.reshape(NPAIR)
    be_flat = be.reshape(CHUNK)[:NB]
    xs = _dispatch(x_flat, pos1)
    ys = _ffn(be_flat, xs, W0, b0, W1, b1, W2, b2)
    yw = _gather_back(ys, pos1)
    out = _combine(yw, w1, w2)
    return out.reshape(bsz, num_pairs, OUT)


# speedup vs baseline: 1.0379x; 1.0379x over previous
import functools

import jax
import jax.numpy as jnp
from jax import lax
from jax.experimental import pallas as pl
from jax.experimental.pallas import tpu as pltpu
from jax.experimental.pallas import tpu_sc as plsc

EXPERTS = 8
D = 768
OUT = 768
N_TOK = 4096
NPAIR = 2 * N_TOK
MB = 512
NB = (NPAIR + EXPERTS * (MB - 1) + MB - 1) // MB
RPAD = NB * MB
CHUNK = 128
N_WORKERS = 32


def _route_kernel(xg_ref, gw_ref, gb_ref, w1_ref, w2_ref, pos_ref, be_ref,
                  ep_ref, cs_ref):
    xg = xg_ref[...]
    gates = lax.dot_general(
        xg, gw_ref[...], (((1,), (1,)), ((), ())),
        preferred_element_type=jnp.float32) + gb_ref[...][None, :]

    idx = lax.broadcasted_iota(jnp.int32, gates.shape, 1)
    v1 = jnp.max(gates, axis=-1, keepdims=True)
    i1 = jnp.min(jnp.where(gates == v1, idx, EXPERTS), axis=-1, keepdims=True)
    masked = jnp.where(idx == i1, -jnp.inf, gates)
    v2 = jnp.max(masked, axis=-1, keepdims=True)
    i2 = jnp.min(jnp.where(masked == v2, idx, EXPERTS), axis=-1, keepdims=True)
    t = jnp.exp(v2 - v1)
    w1_ref[...] = 1.0 / (1.0 + t)
    w2_ref[...] = t / (1.0 + t)

    ep_ref[pl.ds(0, N_TOK), :] = i1
    ep_ref[pl.ds(N_TOK, N_TOK), :] = i2

    csz = 512
    nch = NPAIR // csz
    iota_e = lax.broadcasted_iota(jnp.int32, (csz, EXPERTS), 1)

    ltri = (lax.broadcasted_iota(jnp.int32, (csz, csz), 0) >=
            lax.broadcasted_iota(jnp.int32, (csz, csz), 1)
            ).astype(jnp.float32)
    carry = jnp.zeros((1, EXPERTS), jnp.float32)
    for i in range(nch):
        ch = ep_ref[pl.ds(i * csz, csz), :]
        oh = (ch == iota_e).astype(jnp.float32)
        cs = lax.dot_general(ltri, oh, (((1,), (0,)), ((), ())),
                             preferred_element_type=jnp.float32) + carry
        cs_ref[pl.ds(i * csz, csz), :] = cs
        carry = cs[csz - 1:csz, :]

    counts = carry
    cap = ((counts.astype(jnp.int32) + MB - 1) // MB) * MB
    capf = cap.astype(jnp.float32)
    sl = (lax.broadcasted_iota(jnp.int32, (EXPERTS, EXPERTS), 0) <
          lax.broadcasted_iota(jnp.int32, (EXPERTS, EXPERTS), 1))
    pstartf = lax.dot_general(capf, sl.astype(jnp.float32),
                              (((1,), (0,)), ((), ())),
                              preferred_element_type=jnp.float32)
    pstart = pstartf.astype(jnp.int32)

    for i in range(nch):
        ch = ep_ref[pl.ds(i * csz, csz), :]
        oh = (ch == iota_e).astype(jnp.float32)
        cs = cs_ref[pl.ds(i * csz, csz), :]
        posf = jnp.sum(oh * (pstartf + cs), axis=1, keepdims=True) - 1.0
        pos_ref[pl.ds(i * csz, csz), :] = posf.astype(jnp.int32)

    brow = lax.broadcasted_iota(jnp.int32, (CHUNK, EXPERTS), 0) * MB
    iota_be = lax.broadcasted_iota(jnp.int32, (CHUNK, EXPERTS), 1)
    inseg = (brow >= pstart) & (brow < pstart + cap)
    be = jnp.sum(jnp.where(inseg, iota_be, 0), axis=1, keepdims=True)
    total_used = pstart[:, EXPERTS - 1:] + cap[:, EXPERTS - 1:]
    last_e = jnp.max(jnp.where(cap > 0, iota_be[:1], 0), axis=1,
                     keepdims=True)
    valid = brow[:, :1] < total_used
    be_ref[...] = jnp.where(valid, be, EXPERTS + last_e)


@functools.partial(jax.jit, static_argnames=("interpret",))
def _route(xg, gate_W, gate_b, interpret=False):
    return pl.pallas_call(
        _route_kernel,
        out_shape=[
            jax.ShapeDtypeStruct((N_TOK, 1), jnp.float32),
            jax.ShapeDtypeStruct((N_TOK, 1), jnp.float32),
            jax.ShapeDtypeStruct((NPAIR, 1), jnp.int32),
            jax.ShapeDtypeStruct((CHUNK, 1), jnp.int32),
        ],
        scratch_shapes=[pltpu.VMEM((NPAIR, 1), jnp.int32),
                        pltpu.VMEM((NPAIR, EXPERTS), jnp.float32)],
        interpret=interpret,
    )(xg, gate_W, gate_b)


def _ffn_kernel(be_ref, xs_ref, w0_ref, b0_ref, w1_ref, b1_ref, w2_ref,
                b2_ref, ys_ref):
    @pl.when(be_ref[pl.program_id(0)] < EXPERTS)
    def _():
        x = xs_ref[...]
        h = lax.dot_general(x, w0_ref[0], (((1,), (1,)), ((), ())),
                            preferred_element_type=jnp.float32)
        h = jnp.maximum(h + b0_ref[0], 0.0)
        h = lax.dot_general(h, w1_ref[0], (((1,), (1,)), ((), ())),
                            preferred_element_type=jnp.float32)
        h = jnp.maximum(h + b1_ref[0], 0.0)
        o = lax.dot_general(h, w2_ref[0], (((1,), (1,)), ((), ())),
                            preferred_element_type=jnp.float32)
        ys_ref[...] = o + b2_ref[0]


@functools.partial(jax.jit, static_argnames=("interpret",))
def _ffn(be, xs, W0, b0, W1, b1, W2, b2, interpret=False):
    grid_spec = pltpu.PrefetchScalarGridSpec(
        num_scalar_prefetch=1,
        grid=(NB,),
        in_specs=[
            pl.BlockSpec((MB, D), lambda b, be: (b, 0)),
            pl.BlockSpec((1, D, D), lambda b, be: (be[b] % EXPERTS, 0, 0)),
            pl.BlockSpec((1, 1, D), lambda b, be: (be[b] % EXPERTS, 0, 0)),
            pl.BlockSpec((1, D, D), lambda b, be: (be[b] % EXPERTS, 0, 0)),
            pl.BlockSpec((1, 1, D), lambda b, be: (be[b] % EXPERTS, 0, 0)),
            pl.BlockSpec((1, OUT, D), lambda b, be: (be[b] % EXPERTS, 0, 0)),
            pl.BlockSpec((1, 1, OUT), lambda b, be: (be[b] % EXPERTS, 0, 0)),
        ],
        out_specs=pl.BlockSpec((MB, OUT), lambda b, be: (b, 0)),
    )
    return pl.pallas_call(
        _ffn_kernel,
        grid_spec=grid_spec,
        out_shape=jax.ShapeDtypeStruct((RPAD, OUT), jnp.float32),
        compiler_params=pltpu.CompilerParams(
            dimension_semantics=("arbitrary",)),
        interpret=interpret,
    )(be, xs, W0, b0.reshape(EXPERTS, 1, D), W1, b1.reshape(EXPERTS, 1, D),
      W2, b2.reshape(EXPERTS, 1, OUT))


def _combine_kernel(y1_ref, y2_ref, w1_ref, w2_ref, out_ref):
    out_ref[...] = (y1_ref[...] * w1_ref[...] + y2_ref[...] * w2_ref[...])


@functools.partial(jax.jit, static_argnames=("interpret",))
def _combine(yw, w1, w2, interpret=False):
    mc = 1024
    nbc = N_TOK // mc
    return pl.pallas_call(
        _combine_kernel,
        grid=(nbc,),
        in_specs=[
            pl.BlockSpec((mc, OUT), lambda b: (b, 0)),
            pl.BlockSpec((mc, OUT), lambda b, nbc=nbc: (b + nbc, 0)),
            pl.BlockSpec((mc, 1), lambda b: (b, 0)),
            pl.BlockSpec((mc, 1), lambda b: (b, 0)),
        ],
        out_specs=pl.BlockSpec((mc, OUT), lambda b: (b, 0)),
        out_shape=jax.ShapeDtypeStruct((N_TOK, OUT), jnp.float32),
        interpret=interpret,
    )(yw, yw, w1, w2)


def _sc_mesh():
    return plsc.VectorSubcoreMesh(core_axis_name="c", subcore_axis_name="s")


@jax.jit
def _dispatch(x_flat, pos):
    @functools.partial(
        pl.kernel,
        out_type=jax.ShapeDtypeStruct((RPAD, D), jnp.float32),
        mesh=_sc_mesh(),
        scratch_types=[
            pltpu.VMEM((CHUNK,), jnp.int32),
            pltpu.VMEM((CHUNK, D), jnp.float32),
            pltpu.SemaphoreType.DMA,
        ],
    )
    def disp(x_hbm, pos_hbm, xs_hbm, idx_v, buf_v, sem):
        wid = lax.axis_index("s") * 2 + lax.axis_index("c")
        per_w = NPAIR // N_WORKERS

        @pl.loop(0, per_w // CHUNK)
        def _(c):
            p0 = wid * per_w + c * CHUNK
            t0 = lax.rem(p0, N_TOK)
            pltpu.sync_copy(pos_hbm.at[pl.ds(p0, CHUNK)], idx_v)
            pltpu.sync_copy(x_hbm.at[pl.ds(t0, CHUNK)], buf_v)
            pltpu.async_copy(buf_v, xs_hbm.at[idx_v], sem).wait()

    return disp(x_flat, pos)


@jax.jit
def _gather_back(ys, pos):
    @functools.partial(
        pl.kernel,
        out_type=jax.ShapeDtypeStruct((NPAIR, OUT), jnp.float32),
        mesh=_sc_mesh(),
        scratch_types=[
            pltpu.VMEM((CHUNK,), jnp.int32),
            pltpu.VMEM((CHUNK, OUT), jnp.float32),
            pltpu.SemaphoreType.DMA,
        ],
    )
    def gat(ys_hbm, pos_hbm, yw_hbm, idx_v, buf_v, sem):
        wid = lax.axis_index("s") * 2 + lax.axis_index("c")
        per_w = NPAIR // N_WORKERS

        @pl.loop(0, per_w // CHUNK)
        def _(c):
            p0 = wid * per_w + c * CHUNK
            pltpu.sync_copy(pos_hbm.at[pl.ds(p0, CHUNK)], idx_v)
            pltpu.async_copy(ys_hbm.at[idx_v], buf_v, sem).wait()
            pltpu.sync_copy(buf_v, yw_hbm.at[pl.ds(p0, CHUNK)])

    return gat(ys, pos)


def kernel(x, gate_W, gate_b, W0, b0, W1, b1, W2, b2):
    bsz, num_pairs, feat = x.shape
    x_flat = x.reshape(-1, feat)
    xg = x_flat[:, feat - 3:]
    w1, w2, pos, be = _route(xg, gate_W, gate_b)
    pos1 = pos.reshape(NPAIR)
    be_flat = be.reshape(CHUNK)[:NB]
    xs = _dispatch(x_flat, pos1)
    ys = _ffn(be_flat, xs, W0, b0, W1, b1, W2, b2)
    yw = _gather_back(ys, pos1)
    out = _combine(yw, w1, w2)
    return out.reshape(bsz, num_pairs, OUT)


# speedup vs baseline: 1.0650x; 1.0261x over previous
import functools

import jax
import jax.numpy as jnp
from jax import lax
from jax.experimental import pallas as pl
from jax.experimental.pallas import tpu as pltpu
from jax.experimental.pallas import tpu_sc as plsc

EXPERTS = 8
D = 768
OUT = 768
N_TOK = 4096
NPAIR = 2 * N_TOK
MB = 1024
NB = (NPAIR + EXPERTS * (MB - 1) + MB - 1) // MB
RPAD = NB * MB
CHUNK = 128
N_WORKERS = 32


def _route_kernel(xg_ref, gw_ref, gb_ref, w1_ref, w2_ref, pos_ref, be_ref,
                  ep_ref, cs_ref):
    xg = xg_ref[...]
    gates = lax.dot_general(
        xg, gw_ref[...], (((1,), (1,)), ((), ())),
        preferred_element_type=jnp.float32) + gb_ref[...][None, :]

    idx = lax.broadcasted_iota(jnp.int32, gates.shape, 1)
    v1 = jnp.max(gates, axis=-1, keepdims=True)
    i1 = jnp.min(jnp.where(gates == v1, idx, EXPERTS), axis=-1, keepdims=True)
    masked = jnp.where(idx == i1, -jnp.inf, gates)
    v2 = jnp.max(masked, axis=-1, keepdims=True)
    i2 = jnp.min(jnp.where(masked == v2, idx, EXPERTS), axis=-1, keepdims=True)
    t = jnp.exp(v2 - v1)
    w1_ref[...] = 1.0 / (1.0 + t)
    w2_ref[...] = t / (1.0 + t)

    ep_ref[pl.ds(0, N_TOK), :] = i1
    ep_ref[pl.ds(N_TOK, N_TOK), :] = i2

    csz = 512
    nch = NPAIR // csz
    iota_e = lax.broadcasted_iota(jnp.int32, (csz, EXPERTS), 1)

    ltri = (lax.broadcasted_iota(jnp.int32, (csz, csz), 0) >=
            lax.broadcasted_iota(jnp.int32, (csz, csz), 1)
            ).astype(jnp.float32)
    carry = jnp.zeros((1, EXPERTS), jnp.float32)
    for i in range(nch):
        ch = ep_ref[pl.ds(i * csz, csz), :]
        oh = (ch == iota_e).astype(jnp.float32)
        cs = lax.dot_general(ltri, oh, (((1,), (0,)), ((), ())),
                             preferred_element_type=jnp.float32) + carry
        cs_ref[pl.ds(i * csz, csz), :] = cs
        carry = cs[csz - 1:csz, :]

    counts = carry
    cap = ((counts.astype(jnp.int32) + MB - 1) // MB) * MB
    capf = cap.astype(jnp.float32)
    sl = (lax.broadcasted_iota(jnp.int32, (EXPERTS, EXPERTS), 0) <
          lax.broadcasted_iota(jnp.int32, (EXPERTS, EXPERTS), 1))
    pstartf = lax.dot_general(capf, sl.astype(jnp.float32),
                              (((1,), (0,)), ((), ())),
                              preferred_element_type=jnp.float32)
    pstart = pstartf.astype(jnp.int32)

    for i in range(nch):
        ch = ep_ref[pl.ds(i * csz, csz), :]
        oh = (ch == iota_e).astype(jnp.float32)
        cs = cs_ref[pl.ds(i * csz, csz), :]
        posf = jnp.sum(oh * (pstartf + cs), axis=1, keepdims=True) - 1.0
        pos_ref[pl.ds(i * csz, csz), :] = posf.astype(jnp.int32)

    brow = lax.broadcasted_iota(jnp.int32, (CHUNK, EXPERTS), 0) * MB
    iota_be = lax.broadcasted_iota(jnp.int32, (CHUNK, EXPERTS), 1)
    inseg = (brow >= pstart) & (brow < pstart + cap)
    be = jnp.sum(jnp.where(inseg, iota_be, 0), axis=1, keepdims=True)
    total_used = pstart[:, EXPERTS - 1:] + cap[:, EXPERTS - 1:]
    last_e = jnp.max(jnp.where(cap > 0, iota_be[:1], 0), axis=1,
                     keepdims=True)
    valid = brow[:, :1] < total_used
    be_ref[...] = jnp.where(valid, be, EXPERTS + last_e)


@functools.partial(jax.jit, static_argnames=("interpret",))
def _route(xg, gate_W, gate_b, interpret=False):
    return pl.pallas_call(
        _route_kernel,
        out_shape=[
            jax.ShapeDtypeStruct((N_TOK, 1), jnp.float32),
            jax.ShapeDtypeStruct((N_TOK, 1), jnp.float32),
            jax.ShapeDtypeStruct((NPAIR, 1), jnp.int32),
            jax.ShapeDtypeStruct((CHUNK, 1), jnp.int32),
        ],
        scratch_shapes=[pltpu.VMEM((NPAIR, 1), jnp.int32),
                        pltpu.VMEM((NPAIR, EXPERTS), jnp.float32)],
        interpret=interpret,
    )(xg, gate_W, gate_b)


def _ffn_kernel(be_ref, xs_ref, w0_ref, b0_ref, w1_ref, b1_ref, w2_ref,
                b2_ref, ys_ref):
    @pl.when(be_ref[pl.program_id(0)] < EXPERTS)
    def _():
        x = xs_ref[...]
        h = lax.dot_general(x, w0_ref[0], (((1,), (1,)), ((), ())),
                            preferred_element_type=jnp.float32)
        h = jnp.maximum(h + b0_ref[0], 0.0)
        h = lax.dot_general(h, w1_ref[0], (((1,), (1,)), ((), ())),
                            preferred_element_type=jnp.float32)
        h = jnp.maximum(h + b1_ref[0], 0.0)
        o = lax.dot_general(h, w2_ref[0], (((1,), (1,)), ((), ())),
                            preferred_element_type=jnp.float32)
        ys_ref[...] = o + b2_ref[0]


@functools.partial(jax.jit, static_argnames=("interpret",))
def _ffn(be, xs, W0, b0, W1, b1, W2, b2, interpret=False):
    grid_spec = pltpu.PrefetchScalarGridSpec(
        num_scalar_prefetch=1,
        grid=(NB,),
        in_specs=[
            pl.BlockSpec((MB, D), lambda b, be: (b, 0)),
            pl.BlockSpec((1, D, D), lambda b, be: (be[b] % EXPERTS, 0, 0)),
            pl.BlockSpec((1, 1, D), lambda b, be: (be[b] % EXPERTS, 0, 0)),
            pl.BlockSpec((1, D, D), lambda b, be: (be[b] % EXPERTS, 0, 0)),
            pl.BlockSpec((1, 1, D), lambda b, be: (be[b] % EXPERTS, 0, 0)),
            pl.BlockSpec((1, OUT, D), lambda b, be: (be[b] % EXPERTS, 0, 0)),
            pl.BlockSpec((1, 1, OUT), lambda b, be: (be[b] % EXPERTS, 0, 0)),
        ],
        out_specs=pl.BlockSpec((MB, OUT), lambda b, be: (b, 0)),
    )
    return pl.pallas_call(
        _ffn_kernel,
        grid_spec=grid_spec,
        out_shape=jax.ShapeDtypeStruct((RPAD, OUT), jnp.float32),
        compiler_params=pltpu.CompilerParams(
            dimension_semantics=("arbitrary",)),
        interpret=interpret,
    )(be, xs, W0, b0.reshape(EXPERTS, 1, D), W1, b1.reshape(EXPERTS, 1, D),
      W2, b2.reshape(EXPERTS, 1, OUT))


def _combine_kernel(y1_ref, y2_ref, w1_ref, w2_ref, out_ref):
    out_ref[...] = (y1_ref[...] * w1_ref[...] + y2_ref[...] * w2_ref[...])


@functools.partial(jax.jit, static_argnames=("interpret",))
def _combine(yw, w1, w2, interpret=False):
    mc = 1024
    nbc = N_TOK // mc
    return pl.pallas_call(
        _combine_kernel,
        grid=(nbc,),
        in_specs=[
            pl.BlockSpec((mc, OUT), lambda b: (b, 0)),
            pl.BlockSpec((mc, OUT), lambda b, nbc=nbc: (b + nbc, 0)),
            pl.BlockSpec((mc, 1), lambda b: (b, 0)),
            pl.BlockSpec((mc, 1), lambda b: (b, 0)),
        ],
        out_specs=pl.BlockSpec((mc, OUT), lambda b: (b, 0)),
        out_shape=jax.ShapeDtypeStruct((N_TOK, OUT), jnp.float32),
        interpret=interpret,
    )(yw, yw, w1, w2)


def _sc_mesh():
    return plsc.VectorSubcoreMesh(core_axis_name="c", subcore_axis_name="s")


@jax.jit
def _dispatch(x_flat, pos):
    @functools.partial(
        pl.kernel,
        out_type=jax.ShapeDtypeStruct((RPAD, D), jnp.float32),
        mesh=_sc_mesh(),
        scratch_types=[
            pltpu.VMEM((CHUNK,), jnp.int32),
            pltpu.VMEM((CHUNK, D), jnp.float32),
            pltpu.SemaphoreType.DMA,
        ],
    )
    def disp(x_hbm, pos_hbm, xs_hbm, idx_v, buf_v, sem):
        wid = lax.axis_index("s") * 2 + lax.axis_index("c")
        per_w = NPAIR // N_WORKERS

        @pl.loop(0, per_w // CHUNK)
        def _(c):
            p0 = wid * per_w + c * CHUNK
            t0 = lax.rem(p0, N_TOK)
            pltpu.sync_copy(pos_hbm.at[pl.ds(p0, CHUNK)], idx_v)
            pltpu.sync_copy(x_hbm.at[pl.ds(t0, CHUNK)], buf_v)
            pltpu.async_copy(buf_v, xs_hbm.at[idx_v], sem).wait()

    return disp(x_flat, pos)


@jax.jit
def _gather_back(ys, pos):
    @functools.partial(
        pl.kernel,
        out_type=jax.ShapeDtypeStruct((NPAIR, OUT), jnp.float32),
        mesh=_sc_mesh(),
        scratch_types=[
            pltpu.VMEM((CHUNK,), jnp.int32),
            pltpu.VMEM((CHUNK, OUT), jnp.float32),
            pltpu.SemaphoreType.DMA,
        ],
    )
    def gat(ys_hbm, pos_hbm, yw_hbm, idx_v, buf_v, sem):
        wid = lax.axis_index("s") * 2 + lax.axis_index("c")
        per_w = NPAIR // N_WORKERS

        @pl.loop(0, per_w // CHUNK)
        def _(c):
            p0 = wid * per_w + c * CHUNK
            pltpu.sync_copy(pos_hbm.at[pl.ds(p0, CHUNK)], idx_v)
            pltpu.async_copy(ys_hbm.at[idx_v], buf_v, sem).wait()
            pltpu.sync_copy(buf_v, yw_hbm.at[pl.ds(p0, CHUNK)])

    return gat(ys, pos)


def kernel(x, gate_W, gate_b, W0, b0, W1, b1, W2, b2):
    bsz, num_pairs, feat = x.shape
    x_flat = x.reshape(-1, feat)
    xg = x_flat[:, feat - 3:]
    w1, w2, pos, be = _route(xg, gate_W, gate_b)
    pos1 = pos.reshape(NPAIR)
    be_flat = be.reshape(CHUNK)[:NB]
    xs = _dispatch(x_flat, pos1)
    ys = _ffn(be_flat, xs, W0, b0, W1, b1, W2, b2)
    yw = _gather_back(ys, pos1)
    out = _combine(yw, w1, w2)
    return out.reshape(bsz, num_pairs, OUT)
